# baseline (speedup 1.0000x reference)
import functools

import jax
import jax.numpy as jnp
from jax import lax
from jax.experimental import pallas as pl
from jax.experimental.pallas import tpu as pltpu, tpu_sc as plsc

EB = 320000
NA = 10000
NB = 32
BF = 144
H = 128
NMOL = 100
DEPTH = 4

NW = 32
PNA = 10240
APW = PNA // NW
A_CHUNK = 4
GS_CHUNKS = APW // A_CHUNK
GS_NBUF = 4

_MESH = plsc.VectorSubcoreMesh(core_axis_name="c", subcore_axis_name="s")


def _init_body(x_ref, w_ref, inp_ref, msg_ref):
    inp = jnp.dot(x_ref[...], w_ref[...], preferred_element_type=jnp.float32)
    inp_ref[...] = inp
    msg_ref[...] = jnp.maximum(inp, 0.0)


def _tc_init(f_bonds, w_i):
    rb = 2000
    return pl.pallas_call(
        _init_body,
        grid=(EB // rb,),
        in_specs=[pl.BlockSpec((rb, BF), lambda b: (b, 0)),
                  pl.BlockSpec((BF, H), lambda b: (0, 0))],
        out_specs=[pl.BlockSpec((rb, H), lambda b: (b, 0)),
                   pl.BlockSpec((rb, H), lambda b: (b, 0))],
        out_shape=[jax.ShapeDtypeStruct((EB, H), jnp.float32),
                   jax.ShapeDtypeStruct((EB, H), jnp.float32)],
    )(f_bonds, w_i)


def _update_body(inp_ref, gam_ref, grev_ref, w_ref, o_ref):
    m = gam_ref[...] - grev_ref[...]
    msg = jnp.maximum(
        inp_ref[...] + jnp.dot(m, w_ref[...], preferred_element_type=jnp.float32),
        0.0)
    o_ref[...] = msg


def _tc_update(inp, g_am, g_rev, w_h):
    rb = 2000
    return pl.pallas_call(
        _update_body,
        grid=(EB // rb,),
        in_specs=[pl.BlockSpec((rb, H), lambda b: (b, 0)),
                  pl.BlockSpec((rb, H), lambda b: (b, 0)),
                  pl.BlockSpec((rb, H), lambda b: (b, 0)),
                  pl.BlockSpec((H, H), lambda b: (0, 0))],
        out_specs=pl.BlockSpec((rb, H), lambda b: (b, 0)),
        out_shape=jax.ShapeDtypeStruct((EB, H), jnp.float32),
    )(inp, g_am, g_rev, w_h)


def _readout_body(fa_ref, am_ref, wo_ref, bo_ref, o_ref):
    wo = wo_ref[...]
    h = jnp.dot(fa_ref[...], wo[:H], preferred_element_type=jnp.float32)
    h = h + jnp.dot(am_ref[...], wo[H:],
                    preferred_element_type=jnp.float32)
    h = jnp.maximum(h + bo_ref[...], 0.0)
    o_ref[...] = jnp.mean(h.reshape(NMOL, NA // NMOL, H), axis=1)


def _tc_readout(f_atoms, a_msg, w_o, b_o):
    return pl.pallas_call(
        _readout_body,
        in_specs=[pl.BlockSpec((NA, H), lambda: (0, 0)),
                  pl.BlockSpec((NA, H), lambda: (0, 0)),
                  pl.BlockSpec((2 * H, H), lambda: (0, 0)),
                  pl.BlockSpec((1, H), lambda: (0, 0))],
        out_specs=pl.BlockSpec((NMOL, H), lambda: (0, 0)),
        out_shape=jax.ShapeDtypeStruct((NMOL, H), jnp.float32),
    )(f_atoms, a_msg, w_o, b_o)


def _sc_dual_gather(msg, a_msg, b2revb, b2a):
    k_rows = 80
    nbuf = 2
    rpw = EB // NW
    nch = rpw // k_rows
    ngrp = nch // nbuf

    @functools.partial(
        pl.kernel,
        out_type=(jax.ShapeDtypeStruct((EB, H), jnp.float32),
                  jax.ShapeDtypeStruct((EB, H), jnp.float32)),
        mesh=_MESH,
        scratch_types=[pltpu.VMEM((rpw,), jnp.int32),
                       pltpu.VMEM((nbuf, k_rows, H), jnp.float32),
                       pltpu.VMEM_SHARED((PNA, H), jnp.float32),
                       pltpu.SemaphoreType.DMA((2,)),
                       pltpu.SemaphoreType.DMA((2,)),
                       pltpu.SemaphoreType.DMA],
    )
    def k(msg_hbm, am_hbm, brev_hbm, b2a_hbm, grev_hbm, gam_hbm,
          idx_v, bufs, am_sh, gsem, wsem, ssem):
        wid = lax.axis_index("s") * 2 + lax.axis_index("c")
        sid = lax.axis_index("s")
        base0 = wid * rpw
        spr = PNA // 16

        stage = pltpu.async_copy(
            am_hbm.at[pl.ds(sid * spr, spr)],
            am_sh.at[pl.ds(sid * spr, spr)], ssem)

        def phase(idx_hbm, table_hbm, out_hbm):
            pltpu.sync_copy(idx_hbm.at[pl.ds(base0, rpw)], idx_v)

            @pl.loop(0, ngrp)
            def _(g):
                for b in range(nbuf):
                    c = g * nbuf + b

                    @pl.when(g > 0)
                    def _():
                        pltpu.make_async_copy(
                            bufs.at[b], out_hbm.at[pl.ds(base0, k_rows)],
                            wsem.at[b]).wait()

                    pltpu.async_copy(
                        table_hbm.at[idx_v.at[pl.ds(c * k_rows, k_rows)]],
                        bufs.at[b], gsem.at[b])
                for b in range(nbuf):
                    c = g * nbuf + b
                    pltpu.make_async_copy(
                        table_hbm.at[idx_v.at[pl.ds(0, k_rows)]],
                        bufs.at[b], gsem.at[b]).wait()
                    pltpu.async_copy(
                        bufs.at[b],
                        out_hbm.at[pl.ds(base0 + c * k_rows, k_rows)], wsem.at[b])

            c_t = ngrp * nbuf
            pltpu.make_async_copy(
                bufs.at[0], out_hbm.at[pl.ds(base0, k_rows)], wsem.at[0]).wait()
            pltpu.async_copy(
                table_hbm.at[idx_v.at[pl.ds(c_t * k_rows, k_rows)]],
                bufs.at[0], gsem.at[0])
            pltpu.make_async_copy(
                table_hbm.at[idx_v.at[pl.ds(0, k_rows)]],
                bufs.at[0], gsem.at[0]).wait()
            pltpu.async_copy(
                bufs.at[0],
                out_hbm.at[pl.ds(base0 + c_t * k_rows, k_rows)], wsem.at[0])
            for b in range(nbuf):
                pltpu.make_async_copy(
                    bufs.at[b], out_hbm.at[pl.ds(base0, k_rows)], wsem.at[b]).wait()

        phase(brev_hbm, msg_hbm, grev_hbm)
        stage.wait()
        plsc.subcore_barrier()
        phase(b2a_hbm, am_sh, gam_hbm)

    return k(msg, a_msg, b2revb, b2a)


def _sc_gathersum(message, a2b_pad):
    ppw = APW * NB
    rows_c = A_CHUNK * NB

    @functools.partial(
        pl.kernel,
        out_type=jax.ShapeDtypeStruct((PNA, H), jnp.float32),
        mesh=_MESH,
        scratch_types=[pltpu.VMEM((ppw,), jnp.int32),
                       pltpu.VMEM((GS_NBUF, rows_c, H), jnp.float32),
                       pltpu.VMEM((GS_NBUF, A_CHUNK, H), jnp.float32),
                       pltpu.SemaphoreType.DMA((GS_NBUF,)),
                       pltpu.SemaphoreType.DMA((GS_NBUF,))],
    )
    def k(msg_hbm, idx_hbm, out_hbm, idx_v, bufs, outc, gsem, wsem):
        wid = lax.axis_index("s") * 2 + lax.axis_index("c")
        abase0 = wid * APW
        pltpu.sync_copy(idx_hbm.at[pl.ds(wid * ppw, ppw)], idx_v)

        def fire(c, b):
            pltpu.async_copy(
                msg_hbm.at[idx_v.at[pl.ds(c * rows_c, rows_c)]],
                bufs.at[b], gsem.at[b])

        for b in range(GS_NBUF - 1):
            fire(b, b)

        @pl.loop(0, GS_CHUNKS // GS_NBUF)
        def _(g):
            for b in range(GS_NBUF):
                c = g * GS_NBUF + b

                @pl.when(c + GS_NBUF - 1 < GS_CHUNKS)
                def _():
                    fire(c + GS_NBUF - 1, (b + GS_NBUF - 1) % GS_NBUF)

                pltpu.make_async_copy(
                    msg_hbm.at[idx_v.at[pl.ds(0, rows_c)]],
                    bufs.at[b], gsem.at[b]).wait()

                @pl.when(c >= GS_NBUF)
                def _():
                    pltpu.make_async_copy(
                        outc.at[b], out_hbm.at[pl.ds(abase0, A_CHUNK)],
                        wsem.at[b]).wait()

                for a in range(A_CHUNK):
                    for j in range(H // 16):
                        acc = bufs[b, a * NB, pl.ds(j * 16, 16)]
                        for kk in range(1, NB):
                            acc = acc + bufs[b, a * NB + kk, pl.ds(j * 16, 16)]
                        outc[b, a, pl.ds(j * 16, 16)] = acc
                pltpu.async_copy(
                    outc.at[b],
                    out_hbm.at[pl.ds(abase0 + c * A_CHUNK, A_CHUNK)], wsem.at[b])

        for b in range(GS_NBUF):
            pltpu.make_async_copy(
                outc.at[b], out_hbm.at[pl.ds(abase0, A_CHUNK)], wsem.at[b]).wait()

    return k(message, a2b_pad)


def _encode(f_atoms, f_bonds, a2b, b2a, b2revb, w_i, w_h, w_o, b_o):
    a2b_flat = a2b.reshape(-1).astype(jnp.int32)
    a2b_pad = jnp.concatenate(
        [a2b_flat, jnp.zeros(((PNA - NA) * NB,), jnp.int32)])
    b2a = b2a.astype(jnp.int32)
    b2revb = b2revb.astype(jnp.int32)
    b_o2 = b_o.reshape(1, H)

    inp, message = _tc_init(f_bonds, w_i)
    for _ in range(DEPTH - 1):
        a_msg = _sc_gathersum(message, a2b_pad)
        g_rev, g_am = _sc_dual_gather(message, a_msg, b2revb, b2a)
        message = _tc_update(inp, g_am, g_rev, w_h)
    a_msg = _sc_gathersum(message, a2b_pad)[:NA]
    return _tc_readout(f_atoms, a_msg, w_o, b_o2)


def kernel(mol_f_atoms, mol_f_bonds, mol_a2b, mol_b2a, mol_b2revb,
           struct_f_atoms, struct_f_bonds, struct_a2b, struct_b2a, struct_b2revb,
           W_i1, W_h1, W_o1, b_o1, W_i2, W_h2, W_o2, b_o2):
    mol_vecs = _encode(mol_f_atoms, mol_f_bonds, mol_a2b, mol_b2a, mol_b2revb,
                       W_i1, W_h1, W_o1, b_o1)
    struct_vecs = _encode(struct_f_atoms, struct_f_bonds, struct_a2b,
                          struct_b2a, struct_b2revb, W_i2, W_h2, W_o2, b_o2)
    return jnp.concatenate([mol_vecs, struct_vecs], axis=1)


# speedup vs baseline: 1.0250x; 1.0250x over previous
import functools

import jax
import jax.numpy as jnp
from jax import lax
from jax.experimental import pallas as pl
from jax.experimental.pallas import tpu as pltpu, tpu_sc as plsc

EB = 320000
NA = 10000
NB = 32
BF = 144
H = 128
NMOL = 100
DEPTH = 4

NW = 32
PNA = 10240
APW = PNA // NW
A_CHUNK = 4
GS_CHUNKS = APW // A_CHUNK
GS_NBUF = 4

_MESH = plsc.VectorSubcoreMesh(core_axis_name="c", subcore_axis_name="s")


def _init_body(x_ref, w_ref, inp_ref, msg_ref):
    inp = jnp.dot(x_ref[...], w_ref[...], preferred_element_type=jnp.float32)
    inp_ref[...] = inp
    msg_ref[...] = jnp.maximum(inp, 0.0)


def _tc_init(f_bonds, w_i):
    rb = 2000
    return pl.pallas_call(
        _init_body,
        grid=(EB // rb,),
        in_specs=[pl.BlockSpec((rb, BF), lambda b: (b, 0)),
                  pl.BlockSpec((BF, H), lambda b: (0, 0))],
        out_specs=[pl.BlockSpec((rb, H), lambda b: (b, 0)),
                   pl.BlockSpec((rb, H), lambda b: (b, 0))],
        out_shape=[jax.ShapeDtypeStruct((EB, H), jnp.float32),
                   jax.ShapeDtypeStruct((EB, H), jnp.float32)],
    )(f_bonds, w_i)


def _update_body(inp_ref, gam_ref, grev_ref, w_ref, o_ref):
    m = gam_ref[...] - grev_ref[...]
    msg = jnp.maximum(
        inp_ref[...] + jnp.dot(m, w_ref[...], preferred_element_type=jnp.float32),
        0.0)
    o_ref[...] = msg


def _tc_update(inp, g_am, g_rev, w_h):
    rb = 2000
    return pl.pallas_call(
        _update_body,
        grid=(EB // rb,),
        in_specs=[pl.BlockSpec((rb, H), lambda b: (b, 0)),
                  pl.BlockSpec((rb, H), lambda b: (b, 0)),
                  pl.BlockSpec((rb, H), lambda b: (b, 0)),
                  pl.BlockSpec((H, H), lambda b: (0, 0))],
        out_specs=pl.BlockSpec((rb, H), lambda b: (b, 0)),
        out_shape=jax.ShapeDtypeStruct((EB, H), jnp.float32),
    )(inp, g_am, g_rev, w_h)


def _readout_body(fa_ref, am_ref, wo_ref, bo_ref, o_ref):
    wo = wo_ref[...]
    h = jnp.dot(fa_ref[...], wo[:H], preferred_element_type=jnp.float32)
    h = h + jnp.dot(am_ref[...], wo[H:],
                    preferred_element_type=jnp.float32)
    h = jnp.maximum(h + bo_ref[...], 0.0)
    o_ref[...] = jnp.mean(h.reshape(NMOL, NA // NMOL, H), axis=1)


def _tc_readout(f_atoms, a_msg, w_o, b_o):
    return pl.pallas_call(
        _readout_body,
        in_specs=[pl.BlockSpec((NA, H), lambda: (0, 0)),
                  pl.BlockSpec((NA, H), lambda: (0, 0)),
                  pl.BlockSpec((2 * H, H), lambda: (0, 0)),
                  pl.BlockSpec((1, H), lambda: (0, 0))],
        out_specs=pl.BlockSpec((NMOL, H), lambda: (0, 0)),
        out_shape=jax.ShapeDtypeStruct((NMOL, H), jnp.float32),
    )(f_atoms, a_msg, w_o, b_o)


def _sc_dual_gather(msg, a_msg, b2revb, b2a):
    k_rows = 80
    nbuf = 2
    rpw = EB // NW
    nch = rpw // k_rows
    ngrp = nch // nbuf

    @functools.partial(
        pl.kernel,
        out_type=(jax.ShapeDtypeStruct((EB, H), jnp.float32),
                  jax.ShapeDtypeStruct((EB, H), jnp.float32)),
        mesh=_MESH,
        scratch_types=[pltpu.VMEM((rpw,), jnp.int32),
                       pltpu.VMEM((nbuf, k_rows, H), jnp.float32),
                       pltpu.VMEM_SHARED((PNA, H), jnp.float32),
                       pltpu.SemaphoreType.DMA,
                       pltpu.SemaphoreType.DMA,
                       pltpu.SemaphoreType.DMA],
    )
    def k(msg_hbm, am_hbm, brev_hbm, b2a_hbm, grev_hbm, gam_hbm,
          idx_v, bufs, am_sh, gsem, wsem, ssem):
        wid = lax.axis_index("s") * 2 + lax.axis_index("c")
        sid = lax.axis_index("s")
        base0 = wid * rpw
        spr = PNA // 16

        stage = pltpu.async_copy(
            am_hbm.at[pl.ds(sid * spr, spr)],
            am_sh.at[pl.ds(sid * spr, spr)], ssem)

        def phase(idx_hbm, table_hbm, out_hbm):
            pltpu.sync_copy(idx_hbm.at[pl.ds(base0, rpw)], idx_v)

            @pl.loop(0, ngrp)
            def _(g):
                for b in range(nbuf):
                    c = g * nbuf + b

                    @pl.when(g > 0)
                    def _():
                        pltpu.make_async_copy(
                            bufs.at[b], out_hbm.at[pl.ds(base0, k_rows)],
                            wsem).wait()

                    pltpu.async_copy(
                        table_hbm.at[idx_v.at[pl.ds(c * k_rows, k_rows)]],
                        bufs.at[b], gsem)
                for b in range(nbuf):
                    c = g * nbuf + b
                    pltpu.make_async_copy(
                        table_hbm.at[idx_v.at[pl.ds(0, k_rows)]],
                        bufs.at[b], gsem).wait()
                    pltpu.async_copy(
                        bufs.at[b],
                        out_hbm.at[pl.ds(base0 + c * k_rows, k_rows)], wsem)

            c_t = ngrp * nbuf
            pltpu.make_async_copy(
                bufs.at[0], out_hbm.at[pl.ds(base0, k_rows)], wsem).wait()
            pltpu.async_copy(
                table_hbm.at[idx_v.at[pl.ds(c_t * k_rows, k_rows)]],
                bufs.at[0], gsem)
            pltpu.make_async_copy(
                table_hbm.at[idx_v.at[pl.ds(0, k_rows)]],
                bufs.at[0], gsem).wait()
            pltpu.async_copy(
                bufs.at[0],
                out_hbm.at[pl.ds(base0 + c_t * k_rows, k_rows)], wsem)
            for b in range(nbuf):
                pltpu.make_async_copy(
                    bufs.at[b], out_hbm.at[pl.ds(base0, k_rows)], wsem).wait()

        phase(brev_hbm, msg_hbm, grev_hbm)
        stage.wait()
        plsc.subcore_barrier()
        phase(b2a_hbm, am_sh, gam_hbm)

    return k(msg, a_msg, b2revb, b2a)


def _sc_gathersum(message, a2b_pad):
    ppw = APW * NB
    rows_c = A_CHUNK * NB

    @functools.partial(
        pl.kernel,
        out_type=jax.ShapeDtypeStruct((PNA, H), jnp.float32),
        mesh=_MESH,
        scratch_types=[pltpu.VMEM((ppw,), jnp.int32),
                       pltpu.VMEM((GS_NBUF, rows_c, H), jnp.float32),
                       pltpu.VMEM((GS_NBUF, A_CHUNK, H), jnp.float32),
                       pltpu.SemaphoreType.DMA,
                       pltpu.SemaphoreType.DMA],
    )
    def k(msg_hbm, idx_hbm, out_hbm, idx_v, bufs, outc, gsem, wsem):
        wid = lax.axis_index("s") * 2 + lax.axis_index("c")
        abase0 = wid * APW
        pltpu.sync_copy(idx_hbm.at[pl.ds(wid * ppw, ppw)], idx_v)

        def fire(c, b):
            pltpu.async_copy(
                msg_hbm.at[idx_v.at[pl.ds(c * rows_c, rows_c)]],
                bufs.at[b], gsem)

        for b in range(GS_NBUF - 1):
            fire(b, b)

        @pl.loop(0, GS_CHUNKS // GS_NBUF)
        def _(g):
            for b in range(GS_NBUF):
                c = g * GS_NBUF + b

                @pl.when(c + GS_NBUF - 1 < GS_CHUNKS)
                def _():
                    fire(c + GS_NBUF - 1, (b + GS_NBUF - 1) % GS_NBUF)

                pltpu.make_async_copy(
                    msg_hbm.at[idx_v.at[pl.ds(0, rows_c)]],
                    bufs.at[b], gsem).wait()

                @pl.when(c >= GS_NBUF)
                def _():
                    pltpu.make_async_copy(
                        outc.at[b], out_hbm.at[pl.ds(abase0, A_CHUNK)],
                        wsem).wait()

                for a in range(A_CHUNK):
                    for j in range(H // 16):
                        acc = bufs[b, a * NB, pl.ds(j * 16, 16)]
                        for kk in range(1, NB):
                            acc = acc + bufs[b, a * NB + kk, pl.ds(j * 16, 16)]
                        outc[b, a, pl.ds(j * 16, 16)] = acc
                pltpu.async_copy(
                    outc.at[b],
                    out_hbm.at[pl.ds(abase0 + c * A_CHUNK, A_CHUNK)], wsem)

        for b in range(GS_NBUF):
            pltpu.make_async_copy(
                outc.at[b], out_hbm.at[pl.ds(abase0, A_CHUNK)], wsem).wait()

    return k(message, a2b_pad)


def _encode(f_atoms, f_bonds, a2b, b2a, b2revb, w_i, w_h, w_o, b_o):
    a2b_flat = a2b.reshape(-1).astype(jnp.int32)
    a2b_pad = jnp.concatenate(
        [a2b_flat, jnp.zeros(((PNA - NA) * NB,), jnp.int32)])
    b2a = b2a.astype(jnp.int32)
    b2revb = b2revb.astype(jnp.int32)
    b_o2 = b_o.reshape(1, H)

    inp, message = _tc_init(f_bonds, w_i)
    for _ in range(DEPTH - 1):
        a_msg = _sc_gathersum(message, a2b_pad)
        g_rev, g_am = _sc_dual_gather(message, a_msg, b2revb, b2a)
        message = _tc_update(inp, g_am, g_rev, w_h)
    a_msg = _sc_gathersum(message, a2b_pad)[:NA]
    return _tc_readout(f_atoms, a_msg, w_o, b_o2)


def kernel(mol_f_atoms, mol_f_bonds, mol_a2b, mol_b2a, mol_b2revb,
           struct_f_atoms, struct_f_bonds, struct_a2b, struct_b2a, struct_b2revb,
           W_i1, W_h1, W_o1, b_o1, W_i2, W_h2, W_o2, b_o2):
    mol_vecs = _encode(mol_f_atoms, mol_f_bonds, mol_a2b, mol_b2a, mol_b2revb,
                       W_i1, W_h1, W_o1, b_o1)
    struct_vecs = _encode(struct_f_atoms, struct_f_bonds, struct_a2b,
                          struct_b2a, struct_b2revb, W_i2, W_h2, W_o2, b_o2)
    return jnp.concatenate([mol_vecs, struct_vecs], axis=1)
